# Initial kernel scaffold; baseline (speedup 1.0000x reference)
import jax
import jax.numpy as jnp
from jax.experimental import pallas as pl


def kernel(idx, emb_weight, lin_w, lin_b):
    raise NotImplementedError("write your pallas kernel here")


# speedup vs baseline: 6.4770x; 6.4770x over previous
import functools

import jax
import jax.numpy as jnp
from jax import lax
from jax.experimental import pallas as pl

N = 10000
NPAD = 10240
D = 128
K = 10
ALPHA = 3.0
NEG = -3e38
BIGI = 2**30

RB = 256
OB = 400


def _transform_body(x_ref, w_ref, b_ref, t_ref):
    x = x_ref[...]
    w = w_ref[...]
    b = b_ref[...]
    y = lax.dot_general(x, w, (((1,), (1,)), ((), ())),
                        preferred_element_type=jnp.float32)
    t_ref[...] = jnp.tanh(ALPHA * (y + b))


def _topk_body(t_blk_ref, t_all_ref, out_ref):
    t_blk = t_blk_ref[...]
    t_all = t_all_ref[...]
    sim = lax.dot_general(t_blk, t_all, (((1,), (1,)), ((), ())),
                          preferred_element_type=jnp.float32)
    ci = lax.broadcasted_iota(jnp.int32, (RB, NPAD), 1)
    sim = jnp.where(ci < N, sim, NEG)
    rows = []
    for _ in range(K):
        m = jnp.max(sim, axis=1, keepdims=True)
        idx = jnp.min(jnp.where(sim == m, ci, BIGI), axis=1)
        rows.append(idx)
        sim = jnp.where(ci == idx[:, None], NEG, sim)
    rows = rows + [rows[-1]] * (16 - K)
    out_ref[...] = jnp.stack(rows)


def _onehot_body(idx_ref, h_ref):
    r0 = pl.program_id(0) * OB
    ri = lax.broadcasted_iota(jnp.int32, (OB, N), 0) + r0
    acc = jnp.zeros((OB, N), dtype=jnp.float32)
    for k in range(K):
        idxk = idx_ref[k, :N]
        acc = jnp.maximum(acc, jnp.where(ri == idxk[None, :], 1.0, 0.0))
    h_ref[...] = acc


@jax.jit
def kernel(idx, emb_weight, lin_w, lin_b):
    x = jnp.take(emb_weight, idx, axis=0)
    x = jnp.pad(x, ((0, NPAD - N), (0, 0)))
    b2 = lin_b.reshape(1, D)

    t_all = pl.pallas_call(
        _transform_body,
        out_shape=jax.ShapeDtypeStruct((NPAD, D), jnp.float32),
    )(x, lin_w, b2)

    top_idx = pl.pallas_call(
        _topk_body,
        grid=(NPAD // RB,),
        in_specs=[
            pl.BlockSpec((RB, D), lambda i: (i, 0)),
            pl.BlockSpec((NPAD, D), lambda i: (0, 0)),
        ],
        out_specs=pl.BlockSpec((16, RB), lambda i: (0, i)),
        out_shape=jax.ShapeDtypeStruct((16, NPAD), jnp.int32),
    )(t_all, t_all)

    h_t = pl.pallas_call(
        _onehot_body,
        grid=(N // OB,),
        in_specs=[pl.BlockSpec((16, NPAD), lambda i: (0, 0))],
        out_specs=pl.BlockSpec((OB, N), lambda i: (i, 0)),
        out_shape=jax.ShapeDtypeStruct((N, N), jnp.float32),
    )(top_idx)

    return h_t
